# baseline (speedup 1.0000x reference)
import functools

import jax
import jax.numpy as jnp
from jax import lax
from jax.experimental import pallas as pl
from jax.experimental.pallas import tpu as pltpu
from jax.experimental.pallas import tpu_sc as plsc

_NUM_WORKERS = 32
_CHUNK = 128
_K = 4


def _pad_body(t_ref, o_ref):
    o_ref[:, : t_ref.shape[1]] = t_ref[...]


def _tc_pad(table, dp):
    v, d = table.shape
    blk = 10000
    return pl.pallas_call(
        _pad_body,
        grid=(v // blk,),
        in_specs=[pl.BlockSpec((blk, d), lambda i: (i, 0))],
        out_specs=pl.BlockSpec((blk, dp), lambda i: (i, 0)),
        out_shape=jax.ShapeDtypeStruct((v, dp), jnp.float32),
    )(table)


def _narrow_body(h_ref, o_ref):
    o_ref[...] = h_ref[:, : o_ref.shape[1]]


def _narrow_alias_body(h_ref, prev_ref, o_ref):
    del prev_ref
    o_ref[...] = h_ref[:, : o_ref.shape[1]]


def _tc_narrow(h, prev, total, d, block_off):
    m, dp = h.shape
    blk = 6400
    out_sds = jax.ShapeDtypeStruct((total, d), jnp.float32)
    h_spec = pl.BlockSpec((blk, dp), lambda i: (i, 0))
    o_spec = pl.BlockSpec((blk, d), lambda i: (i + block_off, 0))
    if prev is None:
        return pl.pallas_call(
            _narrow_body,
            grid=(m // blk,),
            in_specs=[h_spec],
            out_specs=o_spec,
            out_shape=out_sds,
        )(h)
    return pl.pallas_call(
        _narrow_alias_body,
        grid=(m // blk,),
        in_specs=[h_spec, pl.BlockSpec(memory_space=pl.ANY)],
        out_specs=o_spec,
        out_shape=out_sds,
        input_output_aliases={1: 0},
    )(h, prev)


def _sc_gather(idx3, table):
    n_workers, n_chunks, chunk = idx3.shape
    _, dp = table.shape
    mesh = plsc.VectorSubcoreMesh(core_axis_name="c", subcore_axis_name="s")

    @functools.partial(
        pl.kernel,
        mesh=mesh,
        out_type=jax.ShapeDtypeStruct((n_workers * n_chunks * chunk, dp),
                                      jnp.float32),
        scratch_types=[
            pltpu.VMEM((n_chunks, chunk), jnp.int32),
            pltpu.VMEM((chunk, dp), jnp.float32),
            pltpu.VMEM((chunk, dp), jnp.float32),
            pltpu.SemaphoreType.DMA,
            pltpu.SemaphoreType.DMA,
        ],
    )
    def k(x_hbm, tbl_hbm, out_hbm, idx_v, buf_a, buf_b, sem_a, sem_b):
        wid = lax.axis_index("s") * 2 + lax.axis_index("c")
        base = wid * (n_chunks * chunk)
        pltpu.sync_copy(x_hbm.at[wid], idx_v)
        pltpu.async_copy(tbl_hbm.at[idx_v.at[0]], buf_a, sem_a)

        def body(i, carry):
            g = 2 * i
            pltpu.async_copy(tbl_hbm.at[idx_v.at[g + 1]], buf_b, sem_b)
            pltpu.make_async_copy(tbl_hbm.at[idx_v.at[g]], buf_a,
                                  sem_a).wait()
            pltpu.sync_copy(buf_a, out_hbm.at[pl.ds(base + g * chunk, chunk)])

            @pl.when(g + 2 < n_chunks)
            def _():
                pltpu.async_copy(tbl_hbm.at[idx_v.at[g + 2]], buf_a, sem_a)

            pltpu.make_async_copy(tbl_hbm.at[idx_v.at[g + 1]], buf_b,
                                  sem_b).wait()
            pltpu.sync_copy(buf_b,
                            out_hbm.at[pl.ds(base + (g + 1) * chunk, chunk)])
            return carry

        lax.fori_loop(0, n_chunks // 2, body, 0)

    return k(idx3, table)


def kernel(X, table):
    b, t = X.shape
    _, d = table.shape
    total = b * t
    stripe = total // _K
    n_chunks = stripe // (_NUM_WORKERS * _CHUNK)
    idx4 = X.reshape(_K, _NUM_WORKERS, n_chunks, _CHUNK).astype(jnp.int32)
    table_p = _tc_pad(table, 128)
    out = None
    for k_i in range(_K):
        h = _sc_gather(idx4[k_i], table_p)
        out = _tc_narrow(h, out, total, d, k_i * (stripe // 6400))
    return out.reshape(b, t, d)


# speedup vs baseline: 1.4225x; 1.4225x over previous
import functools

import jax
import jax.numpy as jnp
from jax import lax
from jax.experimental import pallas as pl
from jax.experimental.pallas import tpu as pltpu
from jax.experimental.pallas import tpu_sc as plsc

_NUM_WORKERS = 32
_CHUNK = 128
_NBUF = 4


def _pad_body(t_ref, o_ref):
    o_ref[:, : t_ref.shape[1]] = t_ref[...]


def _tc_pad(table, dp):
    v, d = table.shape
    blk = 10000
    return pl.pallas_call(
        _pad_body,
        grid=(v // blk,),
        in_specs=[pl.BlockSpec((blk, d), lambda i: (i, 0))],
        out_specs=pl.BlockSpec((blk, dp), lambda i: (i, 0)),
        out_shape=jax.ShapeDtypeStruct((v, dp), jnp.float32),
    )(table)


def _sc_gather(idx3, table):
    n_workers, n_chunks, chunk = idx3.shape
    _, dp = table.shape
    mesh = plsc.VectorSubcoreMesh(core_axis_name="c", subcore_axis_name="s")

    @functools.partial(
        pl.kernel,
        mesh=mesh,
        out_type=jax.ShapeDtypeStruct((n_workers * n_chunks * chunk, dp),
                                      jnp.float32),
        scratch_types=[
            pltpu.VMEM((n_chunks, chunk), jnp.int32),
            [pltpu.VMEM((chunk, dp), jnp.float32) for _ in range(_NBUF)],
            [pltpu.SemaphoreType.DMA for _ in range(_NBUF)],
            [pltpu.SemaphoreType.DMA for _ in range(_NBUF)],
        ],
    )
    def k(x_hbm, tbl_hbm, out_hbm, idx_v, bufs, gsems, wsems):
        wid = lax.axis_index("s") * 2 + lax.axis_index("c")
        base = wid * (n_chunks * chunk)

        def out_at(j):
            return out_hbm.at[pl.ds(base + j * chunk, chunk)]

        pltpu.sync_copy(x_hbm.at[wid], idx_v)
        pltpu.async_copy(tbl_hbm.at[idx_v.at[0]], bufs[0], gsems[0])
        pltpu.async_copy(tbl_hbm.at[idx_v.at[1]], bufs[1], gsems[1])

        def body(i, carry):
            for b in range(_NBUF):
                j = _NBUF * i + b
                pltpu.make_async_copy(tbl_hbm.at[idx_v.at[j]], bufs[b],
                                      gsems[b]).wait()
                pltpu.async_copy(bufs[b], out_at(j), wsems[b])
                t = j + 2
                s = (b + 2) % _NBUF

                @pl.when(t < n_chunks)
                def _():
                    @pl.when(t >= _NBUF)
                    def _():
                        pltpu.make_async_copy(bufs[s], out_at(t - _NBUF),
                                              wsems[s]).wait()

                    pltpu.async_copy(tbl_hbm.at[idx_v.at[t]], bufs[s],
                                     gsems[s])
            return carry

        lax.fori_loop(0, n_chunks // _NBUF, body, 0)
        for b in range(_NBUF):
            pltpu.make_async_copy(bufs[b], out_at(n_chunks - _NBUF + b),
                                  wsems[b]).wait()

    return k(idx3, table)


def kernel(X, table):
    b, t = X.shape
    _, d = table.shape
    total = b * t
    n_chunks = total // (_NUM_WORKERS * _CHUNK)
    idx3 = X.reshape(_NUM_WORKERS, n_chunks, _CHUNK).astype(jnp.int32)
    table_p = _tc_pad(table, 128)
    out = _sc_gather(idx3, table_p)
    return out[:, :d].reshape(b, t, d)
